# Initial kernel scaffold; baseline (speedup 1.0000x reference)
import jax
import jax.numpy as jnp
from jax.experimental import pallas as pl


def kernel(x, Wup, Wgate, Wdown, Wpre, Wpost, g_norm, b_norm, Wadapt, adW, adg, adb, Wep, Wop, Wrg, Wre):
    raise NotImplementedError("write your pallas kernel here")


# speedup vs baseline: 2.1524x; 2.1524x over previous
import functools

import jax
import jax.numpy as jnp
from jax.experimental import pallas as pl

B, S, D = 2, 2048, 768
H = 2 * D
A = H // 16
E = 8
G = 4
NG = max(1, E // G)
K = 2
N = B * S
TS = 256


def _silu(x):
    return x * jax.nn.sigmoid(x)


def _ln(x, g, b):
    mu = jnp.mean(x, axis=-1, keepdims=True)
    d = x - mu
    v = jnp.mean(d * d, axis=-1, keepdims=True)
    return d * jax.lax.rsqrt(v + 1e-5) * g + b


def _pre_body(wadapt_ref, wdown_ref, wep_ref, wop_ref, wawd_ref, wepwop_ref):
    wawd_ref[...] = jnp.dot(wadapt_ref[...], wdown_ref[...],
                            preferred_element_type=jnp.float32)
    wepwop_ref[...] = jnp.dot(wep_ref[...], wop_ref[...],
                              preferred_element_type=jnp.float32)


def _main_body(x_ref, wup_ref, wgate_ref, wdown_ref, wpre_ref, wpost_ref,
               gn_ref, bn_ref, wr_ref,
               sh_ref, pre_ref, ai_ref, ao_ref, rlog_ref):
    xt = x_ref[...]
    up = jnp.dot(xt, wup_ref[...], preferred_element_type=jnp.float32)
    gate = jnp.dot(xt, wgate_ref[...], preferred_element_type=jnp.float32)
    hidden = _silu(gate) * up
    sh_ref[...] = jnp.dot(hidden, wdown_ref[...],
                          preferred_element_type=jnp.float32)
    pre = jnp.dot(xt, wpre_ref[...], preferred_element_type=jnp.float32)
    pre_ref[...] = pre
    g = gn_ref[...]
    b = bn_ref[...]
    ai_ref[...] = _ln(pre, g, b)
    po = jnp.dot(hidden, wpost_ref[...], preferred_element_type=jnp.float32)
    ao_ref[...] = _ln(po, g, b)
    rlog_ref[...] = jnp.dot(xt, wr_ref[...], preferred_element_type=jnp.float32)


def _tail_body(sh_ref, pre_ref, aib_ref, aob_ref, rlog_ref,
               adw_ref, adg_ref, adb_ref, wawd_ref, wepwop_ref,
               out_ref):
    i = pl.program_id(1)
    ai_b = aib_ref[0]
    ao_b = aob_ref[0]
    ai_t = aib_ref[0, pl.ds(i * TS, TS), :]

    aw = jax.lax.dot_general(ai_t, ao_b, (((1,), (1,)), ((), ())),
                             preferred_element_type=jnp.float32)
    aw = _silu(jnp.clip(aw, -5.0, 5.0))
    ac = jnp.dot(aw, ai_b, preferred_element_type=jnp.float32)
    adapt_contrib = jnp.dot(ac, wawd_ref[...],
                            preferred_element_type=jnp.float32)
    shared = sh_ref[0] + 0.1 * adapt_contrib

    rl = rlog_ref[0]
    gl0 = rl[:, 0:1]
    gl1 = rl[:, 1:2]
    gm = jnp.maximum(gl0, gl1)
    e0 = jnp.exp(gl0 - gm)
    e1 = jnp.exp(gl1 - gm)
    cw = jnp.maximum(e0, e1) / (e0 + e1)
    chosen = (gl1 > gl0).astype(jnp.int32)

    ll = rl[:, 2:2 + G]
    lm = jnp.max(ll, axis=1, keepdims=True)
    le = jnp.exp(ll - lm)
    lp = le / jnp.sum(le, axis=1, keepdims=True)
    idx4 = jax.lax.broadcasted_iota(jnp.int32, (TS, G), 1)
    p0 = jnp.max(lp, axis=1, keepdims=True)
    ti0 = jnp.min(jnp.where(lp == p0, idx4, G + 1), axis=1, keepdims=True)
    lp2 = jnp.where(idx4 == ti0, -jnp.inf, lp)
    p1 = jnp.max(lp2, axis=1, keepdims=True)
    ti1 = jnp.min(jnp.where(lp2 == p1, idx4, G + 1), axis=1, keepdims=True)
    inv = 1.0 / (p0 + p1 + 1e-7)
    fw0 = cw * p0 * inv
    fw1 = cw * p1 * inv
    wsum = fw0 + fw1

    pre_t = pre_ref[0]
    P = jnp.dot(pre_t, adw_ref[...],
                preferred_element_type=jnp.float32)
    z = jnp.zeros((TS, A), jnp.float32)
    for e in range(E):
        ch = P[:, e * A:(e + 1) * A]
        lnch = _ln(ch, adg_ref[e:e + 1, :], adb_ref[e:e + 1, :])
        g_id = e // G
        l_id = e % G
        w_e = jnp.where(chosen == g_id, 1.0, 0.0) * (
            fw0 * jnp.where(ti0 == l_id, 1.0, 0.0)
            + fw1 * jnp.where(ti1 == l_id, 1.0, 0.0))
        z = z + w_e * lnch
    expert = jnp.dot(z, wepwop_ref[...],
                     preferred_element_type=jnp.float32)
    out_ref[0] = shared * wsum + 0.1 * expert


@functools.partial(jax.jit, static_argnames=("interpret",))
def _run(x, Wup, Wgate, Wdown, Wpre, Wpost, g_norm, b_norm, Wadapt, adW,
         adg, adb, Wep, Wop, Wrg, Wre, interpret=False):
    xf = x.reshape(N, D)
    wr = jnp.concatenate(
        [Wrg, Wre, jnp.zeros((D, 8 - NG - G), jnp.float32)], axis=1)
    adw_all = jnp.transpose(adW, (1, 0, 2)).reshape(A, E * A)
    gn = g_norm.reshape(1, A)
    bn = b_norm.reshape(1, A)

    wawd, wepwop = pl.pallas_call(
        _pre_body,
        out_shape=(jax.ShapeDtypeStruct((A, D), jnp.float32),
                   jax.ShapeDtypeStruct((A, D), jnp.float32)),
        interpret=interpret,
    )(Wadapt, Wdown, Wep, Wop)

    nt = N // TS
    sh, pre, ai, ao, rlog = pl.pallas_call(
        _main_body,
        grid=(nt,),
        in_specs=[
            pl.BlockSpec((TS, D), lambda i: (i, 0)),
            pl.BlockSpec((D, H), lambda i: (0, 0)),
            pl.BlockSpec((D, H), lambda i: (0, 0)),
            pl.BlockSpec((H, D), lambda i: (0, 0)),
            pl.BlockSpec((D, A), lambda i: (0, 0)),
            pl.BlockSpec((H, A), lambda i: (0, 0)),
            pl.BlockSpec((1, A), lambda i: (0, 0)),
            pl.BlockSpec((1, A), lambda i: (0, 0)),
            pl.BlockSpec((D, 8), lambda i: (0, 0)),
        ],
        out_specs=[
            pl.BlockSpec((TS, D), lambda i: (i, 0)),
            pl.BlockSpec((TS, A), lambda i: (i, 0)),
            pl.BlockSpec((TS, A), lambda i: (i, 0)),
            pl.BlockSpec((TS, A), lambda i: (i, 0)),
            pl.BlockSpec((TS, 8), lambda i: (i, 0)),
        ],
        out_shape=(
            jax.ShapeDtypeStruct((N, D), jnp.float32),
            jax.ShapeDtypeStruct((N, A), jnp.float32),
            jax.ShapeDtypeStruct((N, A), jnp.float32),
            jax.ShapeDtypeStruct((N, A), jnp.float32),
            jax.ShapeDtypeStruct((N, 8), jnp.float32),
        ),
        interpret=interpret,
    )(xf, Wup, Wgate, Wdown, Wpre, Wpost, gn, bn, wr)

    st = S // TS
    out = pl.pallas_call(
        _tail_body,
        grid=(B, st),
        in_specs=[
            pl.BlockSpec((1, TS, D), lambda b, i: (b, i, 0)),
            pl.BlockSpec((1, TS, A), lambda b, i: (b, i, 0)),
            pl.BlockSpec((1, S, A), lambda b, i: (b, 0, 0)),
            pl.BlockSpec((1, S, A), lambda b, i: (b, 0, 0)),
            pl.BlockSpec((1, TS, 8), lambda b, i: (b, i, 0)),
            pl.BlockSpec((A, E * A), lambda b, i: (0, 0)),
            pl.BlockSpec((E, A), lambda b, i: (0, 0)),
            pl.BlockSpec((E, A), lambda b, i: (0, 0)),
            pl.BlockSpec((A, D), lambda b, i: (0, 0)),
            pl.BlockSpec((A, D), lambda b, i: (0, 0)),
        ],
        out_specs=pl.BlockSpec((1, TS, D), lambda b, i: (b, i, 0)),
        out_shape=jax.ShapeDtypeStruct((B, S, D), jnp.float32),
        interpret=interpret,
    )(sh.reshape(B, S, D), pre.reshape(B, S, A), ai.reshape(B, S, A),
      ao.reshape(B, S, A), rlog.reshape(B, S, 8),
      adw_all, adg, adb, wawd, wepwop)
    return out


def kernel(x, Wup, Wgate, Wdown, Wpre, Wpost, g_norm, b_norm, Wadapt, adW,
           adg, adb, Wep, Wop, Wrg, Wre):
    return _run(x, Wup, Wgate, Wdown, Wpre, Wpost, g_norm, b_norm, Wadapt,
                adW, adg, adb, Wep, Wop, Wrg, Wre)


# speedup vs baseline: 2.1962x; 1.0203x over previous
import functools

import jax
import jax.numpy as jnp
from jax.experimental import pallas as pl

B, S, D = 2, 2048, 768
H = 2 * D
A = H // 16
E = 8
G = 4
NG = max(1, E // G)
K = 2
N = B * S
TS = 256


def _silu(x):
    return x * jax.nn.sigmoid(x)


def _ln(x, g, b):
    mu = jnp.mean(x, axis=-1, keepdims=True)
    d = x - mu
    v = jnp.mean(d * d, axis=-1, keepdims=True)
    return d * jax.lax.rsqrt(v + 1e-5) * g + b


def _pre_body(wadapt_ref, wdown_ref, wep_ref, wop_ref, wawd_ref, wepwop_ref):
    wawd_ref[...] = jnp.dot(wadapt_ref[...], wdown_ref[...],
                            preferred_element_type=jnp.float32)
    wepwop_ref[...] = jnp.dot(wep_ref[...], wop_ref[...],
                              preferred_element_type=jnp.float32)


def _main_body(x_ref, wup_ref, wgate_ref, wdown_ref, wpre_ref, wpost_ref,
               gn_ref, bn_ref, wr_ref,
               sh_ref, pre_ref, ai_ref, ao_ref, rlog_ref):
    xt = x_ref[...]
    xb = xt.astype(jnp.bfloat16)
    up = jnp.dot(xb, wup_ref[...], preferred_element_type=jnp.float32)
    gate = jnp.dot(xb, wgate_ref[...], preferred_element_type=jnp.float32)
    hidden = (_silu(gate) * up).astype(jnp.bfloat16)
    sh_ref[...] = jnp.dot(hidden, wdown_ref[...],
                          preferred_element_type=jnp.float32)
    pre = jnp.dot(xt, wpre_ref[...], preferred_element_type=jnp.float32)
    pre_ref[...] = pre
    g = gn_ref[...]
    b = bn_ref[...]
    ai_ref[...] = _ln(pre, g, b)
    po = jnp.dot(hidden, wpost_ref[...], preferred_element_type=jnp.float32)
    ao_ref[...] = _ln(po, g, b)
    rlog_ref[...] = jnp.dot(xt, wr_ref[...], preferred_element_type=jnp.float32)


def _tail_body(sh_ref, pre_ref, aib_ref, aob_ref, rlog_ref,
               adw_ref, adg_ref, adb_ref, wawd_ref, wepwop_ref,
               out_ref):
    i = pl.program_id(1)
    ai_b = aib_ref[0]
    ao_b = aob_ref[0]
    ai_t = aib_ref[0, pl.ds(i * TS, TS), :]

    aw = jax.lax.dot_general(ai_t.astype(jnp.bfloat16),
                             ao_b.astype(jnp.bfloat16),
                             (((1,), (1,)), ((), ())),
                             preferred_element_type=jnp.float32)
    aw = _silu(jnp.clip(aw, -5.0, 5.0)).astype(jnp.bfloat16)
    ac = jnp.dot(aw, ai_b.astype(jnp.bfloat16),
                 preferred_element_type=jnp.float32)
    adapt_contrib = jnp.dot(ac, wawd_ref[...],
                            preferred_element_type=jnp.float32)
    shared = sh_ref[0] + 0.1 * adapt_contrib

    rl = rlog_ref[0]
    gl0 = rl[:, 0:1]
    gl1 = rl[:, 1:2]
    gm = jnp.maximum(gl0, gl1)
    e0 = jnp.exp(gl0 - gm)
    e1 = jnp.exp(gl1 - gm)
    cw = jnp.maximum(e0, e1) / (e0 + e1)
    chosen = (gl1 > gl0).astype(jnp.int32)

    ll = rl[:, 2:2 + G]
    lm = jnp.max(ll, axis=1, keepdims=True)
    le = jnp.exp(ll - lm)
    lp = le / jnp.sum(le, axis=1, keepdims=True)
    idx4 = jax.lax.broadcasted_iota(jnp.int32, (TS, G), 1)
    p0 = jnp.max(lp, axis=1, keepdims=True)
    ti0 = jnp.min(jnp.where(lp == p0, idx4, G + 1), axis=1, keepdims=True)
    lp2 = jnp.where(idx4 == ti0, -jnp.inf, lp)
    p1 = jnp.max(lp2, axis=1, keepdims=True)
    ti1 = jnp.min(jnp.where(lp2 == p1, idx4, G + 1), axis=1, keepdims=True)
    inv = 1.0 / (p0 + p1 + 1e-7)
    fw0 = cw * p0 * inv
    fw1 = cw * p1 * inv
    wsum = fw0 + fw1

    pre_t = pre_ref[0]
    P = jnp.dot(pre_t, adw_ref[...],
                preferred_element_type=jnp.float32)
    z = jnp.zeros((TS, A), jnp.float32)
    for e in range(E):
        ch = P[:, e * A:(e + 1) * A]
        lnch = _ln(ch, adg_ref[e:e + 1, :], adb_ref[e:e + 1, :])
        g_id = e // G
        l_id = e % G
        w_e = jnp.where(chosen == g_id, 1.0, 0.0) * (
            fw0 * jnp.where(ti0 == l_id, 1.0, 0.0)
            + fw1 * jnp.where(ti1 == l_id, 1.0, 0.0))
        z = z + w_e * lnch
    expert = jnp.dot(z, wepwop_ref[...],
                     preferred_element_type=jnp.float32)
    out_ref[0] = shared * wsum + 0.1 * expert


@functools.partial(jax.jit, static_argnames=("interpret",))
def _run(x, Wup, Wgate, Wdown, Wpre, Wpost, g_norm, b_norm, Wadapt, adW,
         adg, adb, Wep, Wop, Wrg, Wre, interpret=False):
    xf = x.reshape(N, D)
    wr = jnp.concatenate(
        [Wrg, Wre, jnp.zeros((D, 8 - NG - G), jnp.float32)], axis=1)
    adw_all = jnp.transpose(adW, (1, 0, 2)).reshape(A, E * A)
    gn = g_norm.reshape(1, A)
    bn = b_norm.reshape(1, A)

    wawd, wepwop = pl.pallas_call(
        _pre_body,
        out_shape=(jax.ShapeDtypeStruct((A, D), jnp.float32),
                   jax.ShapeDtypeStruct((A, D), jnp.float32)),
        interpret=interpret,
    )(Wadapt, Wdown, Wep, Wop)

    wup_b = Wup.astype(jnp.bfloat16)
    wgate_b = Wgate.astype(jnp.bfloat16)
    wdown_b = Wdown.astype(jnp.bfloat16)
    wpost_b = Wpost.astype(jnp.bfloat16)

    nt = N // TS
    sh, pre, ai, ao, rlog = pl.pallas_call(
        _main_body,
        grid=(nt,),
        in_specs=[
            pl.BlockSpec((TS, D), lambda i: (i, 0)),
            pl.BlockSpec((D, H), lambda i: (0, 0)),
            pl.BlockSpec((D, H), lambda i: (0, 0)),
            pl.BlockSpec((H, D), lambda i: (0, 0)),
            pl.BlockSpec((D, A), lambda i: (0, 0)),
            pl.BlockSpec((H, A), lambda i: (0, 0)),
            pl.BlockSpec((1, A), lambda i: (0, 0)),
            pl.BlockSpec((1, A), lambda i: (0, 0)),
            pl.BlockSpec((D, 8), lambda i: (0, 0)),
        ],
        out_specs=[
            pl.BlockSpec((TS, D), lambda i: (i, 0)),
            pl.BlockSpec((TS, A), lambda i: (i, 0)),
            pl.BlockSpec((TS, A), lambda i: (i, 0)),
            pl.BlockSpec((TS, A), lambda i: (i, 0)),
            pl.BlockSpec((TS, 8), lambda i: (i, 0)),
        ],
        out_shape=(
            jax.ShapeDtypeStruct((N, D), jnp.float32),
            jax.ShapeDtypeStruct((N, A), jnp.float32),
            jax.ShapeDtypeStruct((N, A), jnp.float32),
            jax.ShapeDtypeStruct((N, A), jnp.float32),
            jax.ShapeDtypeStruct((N, 8), jnp.float32),
        ),
        interpret=interpret,
    )(xf, wup_b, wgate_b, wdown_b, Wpre, wpost_b, gn, bn, wr)

    st = S // TS
    out = pl.pallas_call(
        _tail_body,
        grid=(B, st),
        in_specs=[
            pl.BlockSpec((1, TS, D), lambda b, i: (b, i, 0)),
            pl.BlockSpec((1, TS, A), lambda b, i: (b, i, 0)),
            pl.BlockSpec((1, S, A), lambda b, i: (b, 0, 0)),
            pl.BlockSpec((1, S, A), lambda b, i: (b, 0, 0)),
            pl.BlockSpec((1, TS, 8), lambda b, i: (b, i, 0)),
            pl.BlockSpec((A, E * A), lambda b, i: (0, 0)),
            pl.BlockSpec((E, A), lambda b, i: (0, 0)),
            pl.BlockSpec((E, A), lambda b, i: (0, 0)),
            pl.BlockSpec((A, D), lambda b, i: (0, 0)),
            pl.BlockSpec((A, D), lambda b, i: (0, 0)),
        ],
        out_specs=pl.BlockSpec((1, TS, D), lambda b, i: (b, i, 0)),
        out_shape=jax.ShapeDtypeStruct((B, S, D), jnp.float32),
        interpret=interpret,
    )(sh.reshape(B, S, D), pre.reshape(B, S, A), ai.reshape(B, S, A),
      ao.reshape(B, S, A), rlog.reshape(B, S, 8),
      adw_all, adg, adb, wawd, wepwop)
    return out


def kernel(x, Wup, Wgate, Wdown, Wpre, Wpost, g_norm, b_norm, Wadapt, adW,
           adg, adb, Wep, Wop, Wrg, Wre):
    return _run(x, Wup, Wgate, Wdown, Wpre, Wpost, g_norm, b_norm, Wadapt,
                adW, adg, adb, Wep, Wop, Wrg, Wre)


# speedup vs baseline: 3.2895x; 1.4978x over previous
import functools

import jax
import jax.numpy as jnp
from jax.experimental import pallas as pl

B, S, D = 2, 2048, 768
H = 2 * D
A = H // 16
E = 8
G = 4
NG = max(1, E // G)
K = 2
N = B * S
TS = 256
TT = 512


def _silu(x):
    return x * jax.nn.sigmoid(x)


def _ln(x, g, b):
    mu = jnp.mean(x, axis=-1, keepdims=True)
    d = x - mu
    v = jnp.mean(d * d, axis=-1, keepdims=True)
    return d * jax.lax.rsqrt(v + 1e-5) * g + b


def _pre_body(wadapt_ref, wdown_ref, wep_ref, wop_ref, wawd_ref, wepwop_ref):
    wawd_ref[...] = jnp.dot(wadapt_ref[...], wdown_ref[...],
                            preferred_element_type=jnp.float32)
    wepwop_ref[...] = jnp.dot(wep_ref[...], wop_ref[...],
                              preferred_element_type=jnp.float32)


def _main_body(x_ref, wup_ref, wgate_ref, wdown_ref, wpre_ref, wpost_ref,
               gn_ref, bn_ref, wr_ref,
               sh_ref, pre_ref, ai_ref, ao_ref, rlog_ref):
    xt = x_ref[...]
    xb = xt.astype(jnp.bfloat16)
    up = jnp.dot(xb, wup_ref[...], preferred_element_type=jnp.float32)
    gate = jnp.dot(xb, wgate_ref[...], preferred_element_type=jnp.float32)
    hidden = (_silu(gate) * up).astype(jnp.bfloat16)
    sh_ref[...] = jnp.dot(hidden, wdown_ref[...],
                          preferred_element_type=jnp.float32)
    pre = jnp.dot(xt, wpre_ref[...], preferred_element_type=jnp.float32)
    pre_ref[...] = pre
    g = gn_ref[...]
    b = bn_ref[...]
    ai_ref[...] = _ln(pre, g, b)
    po = jnp.dot(hidden, wpost_ref[...], preferred_element_type=jnp.float32)
    ao_ref[...] = _ln(po, g, b)
    rlog_ref[...] = jnp.dot(xt, wr_ref[...], preferred_element_type=jnp.float32)


def _tail_body(sh_ref, pre_ref, aib_ref, aob_ref, rlog_ref,
               adw_ref, adgf_ref, adb_ref, ub_ref, bb_ref,
               wawd_ref, wepwop_ref, out_ref):
    i = pl.program_id(1)
    ai_b = aib_ref[0]
    ao_b = aob_ref[0]
    ai_t = aib_ref[0, pl.ds(i * TT, TT), :]

    aw = jax.lax.dot_general(ai_t.astype(jnp.bfloat16),
                             ao_b.astype(jnp.bfloat16),
                             (((1,), (1,)), ((), ())),
                             preferred_element_type=jnp.float32)
    aw = _silu(jnp.clip(aw, -5.0, 5.0)).astype(jnp.bfloat16)
    ac = jnp.dot(aw, ai_b.astype(jnp.bfloat16),
                 preferred_element_type=jnp.float32)
    adapt_contrib = jnp.dot(ac, wawd_ref[...],
                            preferred_element_type=jnp.float32)
    shared = sh_ref[0] + 0.1 * adapt_contrib

    rl = rlog_ref[0]
    gl0 = rl[:, 0:1]
    gl1 = rl[:, 1:2]
    gm = jnp.maximum(gl0, gl1)
    e0 = jnp.exp(gl0 - gm)
    e1 = jnp.exp(gl1 - gm)
    cw = jnp.maximum(e0, e1) / (e0 + e1)
    chosen = (gl1 > gl0).astype(jnp.int32)

    ll = rl[:, 2:2 + G]
    lm = jnp.max(ll, axis=1, keepdims=True)
    le = jnp.exp(ll - lm)
    lp = le / jnp.sum(le, axis=1, keepdims=True)
    idx4 = jax.lax.broadcasted_iota(jnp.int32, (TT, G), 1)
    p0 = jnp.max(lp, axis=1, keepdims=True)
    ti0 = jnp.min(jnp.where(lp == p0, idx4, G + 1), axis=1, keepdims=True)
    lp2 = jnp.where(idx4 == ti0, -jnp.inf, lp)
    p1 = jnp.max(lp2, axis=1, keepdims=True)
    ti1 = jnp.min(jnp.where(lp2 == p1, idx4, G + 1), axis=1, keepdims=True)
    inv = 1.0 / (p0 + p1 + 1e-7)
    fw0 = cw * p0 * inv
    fw1 = cw * p1 * inv
    wsum = fw0 + fw1

    g_of_e = jax.lax.broadcasted_iota(jnp.int32, (TT, E), 1) // G
    l_of_e = jax.lax.broadcasted_iota(jnp.int32, (TT, E), 1) % G
    dm = jnp.where(chosen == g_of_e, 1.0, 0.0) * (
        fw0 * jnp.where(ti0 == l_of_e, 1.0, 0.0)
        + fw1 * jnp.where(ti1 == l_of_e, 1.0, 0.0))

    pre_t = pre_ref[0]
    P = jnp.dot(pre_t, adw_ref[...],
                preferred_element_type=jnp.float32)
    sums = jnp.dot(P, ub_ref[...], preferred_element_type=jnp.float32)
    sums2 = jnp.dot(P * P, ub_ref[...], preferred_element_type=jnp.float32)
    mu = sums * (1.0 / A)
    var = sums2 * (1.0 / A) - mu * mu
    rs = jax.lax.rsqrt(var + 1e-5)
    s = dm * rs
    u = s * mu
    Sb = jnp.dot(s, bb_ref[...], preferred_element_type=jnp.float32)
    Ub = jnp.dot(u, bb_ref[...], preferred_element_type=jnp.float32)
    zfull = (P * Sb - Ub) * adgf_ref[...]
    z = zfull[:, 0:A]
    for e in range(1, E):
        z = z + zfull[:, e * A:(e + 1) * A]
    z = z + jnp.dot(dm, adb_ref[...], preferred_element_type=jnp.float32)
    expert = jnp.dot(z, wepwop_ref[...],
                     preferred_element_type=jnp.float32)
    out_ref[0] = shared * wsum + 0.1 * expert


@functools.partial(jax.jit, static_argnames=("interpret",))
def _run(x, Wup, Wgate, Wdown, Wpre, Wpost, g_norm, b_norm, Wadapt, adW,
         adg, adb, Wep, Wop, Wrg, Wre, interpret=False):
    xf = x.reshape(N, D)
    wr = jnp.concatenate(
        [Wrg, Wre, jnp.zeros((D, 8 - NG - G), jnp.float32)], axis=1)
    adw_all = jnp.transpose(adW, (1, 0, 2)).reshape(A, E * A)
    gn = g_norm.reshape(1, A)
    bn = b_norm.reshape(1, A)

    wawd, wepwop = pl.pallas_call(
        _pre_body,
        out_shape=(jax.ShapeDtypeStruct((A, D), jnp.float32),
                   jax.ShapeDtypeStruct((A, D), jnp.float32)),
        interpret=interpret,
    )(Wadapt, Wdown, Wep, Wop)

    wup_b = Wup.astype(jnp.bfloat16)
    wgate_b = Wgate.astype(jnp.bfloat16)
    wdown_b = Wdown.astype(jnp.bfloat16)
    wpost_b = Wpost.astype(jnp.bfloat16)

    nt = N // TS
    sh, pre, ai, ao, rlog = pl.pallas_call(
        _main_body,
        grid=(nt,),
        in_specs=[
            pl.BlockSpec((TS, D), lambda i: (i, 0)),
            pl.BlockSpec((D, H), lambda i: (0, 0)),
            pl.BlockSpec((D, H), lambda i: (0, 0)),
            pl.BlockSpec((H, D), lambda i: (0, 0)),
            pl.BlockSpec((D, A), lambda i: (0, 0)),
            pl.BlockSpec((H, A), lambda i: (0, 0)),
            pl.BlockSpec((1, A), lambda i: (0, 0)),
            pl.BlockSpec((1, A), lambda i: (0, 0)),
            pl.BlockSpec((D, 8), lambda i: (0, 0)),
        ],
        out_specs=[
            pl.BlockSpec((TS, D), lambda i: (i, 0)),
            pl.BlockSpec((TS, A), lambda i: (i, 0)),
            pl.BlockSpec((TS, A), lambda i: (i, 0)),
            pl.BlockSpec((TS, A), lambda i: (i, 0)),
            pl.BlockSpec((TS, 8), lambda i: (i, 0)),
        ],
        out_shape=(
            jax.ShapeDtypeStruct((N, D), jnp.float32),
            jax.ShapeDtypeStruct((N, A), jnp.float32),
            jax.ShapeDtypeStruct((N, A), jnp.float32),
            jax.ShapeDtypeStruct((N, A), jnp.float32),
            jax.ShapeDtypeStruct((N, 8), jnp.float32),
        ),
        interpret=interpret,
    )(xf, wup_b, wgate_b, wdown_b, Wpre, wpost_b, gn, bn, wr)

    ub = jnp.repeat(jnp.eye(E, dtype=jnp.float32), A, axis=0)
    bb = ub.T
    adgf = adg.reshape(1, E * A)

    st = S // TT
    out = pl.pallas_call(
        _tail_body,
        grid=(B, st),
        in_specs=[
            pl.BlockSpec((1, TT, D), lambda b, i: (b, i, 0)),
            pl.BlockSpec((1, TT, A), lambda b, i: (b, i, 0)),
            pl.BlockSpec((1, S, A), lambda b, i: (b, 0, 0)),
            pl.BlockSpec((1, S, A), lambda b, i: (b, 0, 0)),
            pl.BlockSpec((1, TT, 8), lambda b, i: (b, i, 0)),
            pl.BlockSpec((A, E * A), lambda b, i: (0, 0)),
            pl.BlockSpec((1, E * A), lambda b, i: (0, 0)),
            pl.BlockSpec((E, A), lambda b, i: (0, 0)),
            pl.BlockSpec((E * A, E), lambda b, i: (0, 0)),
            pl.BlockSpec((E, E * A), lambda b, i: (0, 0)),
            pl.BlockSpec((A, D), lambda b, i: (0, 0)),
            pl.BlockSpec((A, D), lambda b, i: (0, 0)),
        ],
        out_specs=pl.BlockSpec((1, TT, D), lambda b, i: (b, i, 0)),
        out_shape=jax.ShapeDtypeStruct((B, S, D), jnp.float32),
        interpret=interpret,
    )(sh.reshape(B, S, D), pre.reshape(B, S, A), ai.reshape(B, S, A),
      ao.reshape(B, S, A), rlog.reshape(B, S, 8),
      adw_all, adgf, adb, ub, bb, wawd, wepwop)
    return out


def kernel(x, Wup, Wgate, Wdown, Wpre, Wpost, g_norm, b_norm, Wadapt, adW,
           adg, adb, Wep, Wop, Wrg, Wre):
    return _run(x, Wup, Wgate, Wdown, Wpre, Wpost, g_norm, b_norm, Wadapt,
                adW, adg, adb, Wep, Wop, Wrg, Wre)


# speedup vs baseline: 3.3895x; 1.0304x over previous
import functools

import jax
import jax.numpy as jnp
from jax.experimental import pallas as pl
from jax.experimental.pallas import tpu as pltpu

B, S, D = 2, 2048, 768
H = 2 * D
A = H // 16
E = 8
G = 4
NG = max(1, E // G)
K = 2
N = B * S
TF = 512


def _silu(x):
    return x * jax.nn.sigmoid(x)


def _ln(x, g, b):
    mu = jnp.mean(x, axis=-1, keepdims=True)
    d = x - mu
    v = jnp.mean(d * d, axis=-1, keepdims=True)
    return d * jax.lax.rsqrt(v + 1e-5) * g + b


def _pre_body(wadapt_ref, wdown_ref, wep_ref, wop_ref, wawd_ref, wepwop_ref):
    wawd_ref[...] = jnp.dot(wadapt_ref[...], wdown_ref[...],
                            preferred_element_type=jnp.float32)
    wepwop_ref[...] = jnp.dot(wep_ref[...], wop_ref[...],
                              preferred_element_type=jnp.float32)


def _fused_body(x_ref, wup_ref, wgate_ref, wdown_ref, wpre_ref, wpost_ref,
                gn_ref, bn_ref, wr_ref,
                adw_ref, adgf_ref, adb_ref, ub_ref, bb_ref,
                wawd_ref, wepwop_ref,
                out_ref,
                sh_s, pre_s, ai_s, ao_s, rlog_s):
    p = pl.program_id(1)
    i = pl.program_id(2)
    rows = pl.ds(i * TF, TF)

    @pl.when(p == 0)
    def _main():
        xt = x_ref[0]
        xb = xt.astype(jnp.bfloat16)
        up = jnp.dot(xb, wup_ref[...], preferred_element_type=jnp.float32)
        gate = jnp.dot(xb, wgate_ref[...], preferred_element_type=jnp.float32)
        hidden = (_silu(gate) * up).astype(jnp.bfloat16)
        sh_s[rows, :] = jnp.dot(hidden, wdown_ref[...],
                                preferred_element_type=jnp.float32)
        pre = jnp.dot(xt, wpre_ref[...], preferred_element_type=jnp.float32)
        pre_s[rows, :] = pre
        g = gn_ref[...]
        b = bn_ref[...]
        ai_s[rows, :] = _ln(pre, g, b)
        po = jnp.dot(hidden, wpost_ref[...],
                     preferred_element_type=jnp.float32)
        ao_s[rows, :] = _ln(po, g, b)
        rlog_s[rows, :] = jnp.dot(xt, wr_ref[...],
                                  preferred_element_type=jnp.float32)

    @pl.when(p == 1)
    def _tail():
        ai_b = ai_s[...]
        ao_b = ao_s[...]
        ai_t = ai_s[rows, :]

        aw = jax.lax.dot_general(ai_t.astype(jnp.bfloat16),
                                 ao_b.astype(jnp.bfloat16),
                                 (((1,), (1,)), ((), ())),
                                 preferred_element_type=jnp.float32)
        aw = _silu(jnp.clip(aw, -5.0, 5.0)).astype(jnp.bfloat16)
        ac = jnp.dot(aw, ai_b.astype(jnp.bfloat16),
                     preferred_element_type=jnp.float32)
        adapt_contrib = jnp.dot(ac, wawd_ref[...],
                                preferred_element_type=jnp.float32)
        shared = sh_s[rows, :] + 0.1 * adapt_contrib

        rl = rlog_s[rows, :]
        gl0 = rl[:, 0:1]
        gl1 = rl[:, 1:2]
        gm = jnp.maximum(gl0, gl1)
        e0 = jnp.exp(gl0 - gm)
        e1 = jnp.exp(gl1 - gm)
        cw = jnp.maximum(e0, e1) / (e0 + e1)
        chosen = (gl1 > gl0).astype(jnp.int32)

        ll = rl[:, 2:2 + G]
        lm = jnp.max(ll, axis=1, keepdims=True)
        le = jnp.exp(ll - lm)
        lp = le / jnp.sum(le, axis=1, keepdims=True)
        idx4 = jax.lax.broadcasted_iota(jnp.int32, (TF, G), 1)
        p0 = jnp.max(lp, axis=1, keepdims=True)
        ti0 = jnp.min(jnp.where(lp == p0, idx4, G + 1), axis=1, keepdims=True)
        lp2 = jnp.where(idx4 == ti0, -jnp.inf, lp)
        p1 = jnp.max(lp2, axis=1, keepdims=True)
        ti1 = jnp.min(jnp.where(lp2 == p1, idx4, G + 1), axis=1, keepdims=True)
        inv = 1.0 / (p0 + p1 + 1e-7)
        fw0 = cw * p0 * inv
        fw1 = cw * p1 * inv
        wsum = fw0 + fw1

        g_of_e = jax.lax.broadcasted_iota(jnp.int32, (TF, E), 1) // G
        l_of_e = jax.lax.broadcasted_iota(jnp.int32, (TF, E), 1) % G
        dm = jnp.where(chosen == g_of_e, 1.0, 0.0) * (
            fw0 * jnp.where(ti0 == l_of_e, 1.0, 0.0)
            + fw1 * jnp.where(ti1 == l_of_e, 1.0, 0.0))

        pre_t = pre_s[rows, :]
        P = jnp.dot(pre_t, adw_ref[...],
                    preferred_element_type=jnp.float32)
        sums = jnp.dot(P, ub_ref[...], preferred_element_type=jnp.float32)
        sums2 = jnp.dot(P * P, ub_ref[...],
                        preferred_element_type=jnp.float32)
        mu = sums * (1.0 / A)
        var = sums2 * (1.0 / A) - mu * mu
        rs = jax.lax.rsqrt(var + 1e-5)
        sc = dm * rs
        uc = sc * mu
        Sb = jnp.dot(sc, bb_ref[...], preferred_element_type=jnp.float32)
        Ub = jnp.dot(uc, bb_ref[...], preferred_element_type=jnp.float32)
        zfull = (P * Sb - Ub) * adgf_ref[...]
        z = zfull[:, 0:A]
        for e in range(1, E):
            z = z + zfull[:, e * A:(e + 1) * A]
        z = z + jnp.dot(dm, adb_ref[...], preferred_element_type=jnp.float32)
        expert = jnp.dot(z, wepwop_ref[...],
                         preferred_element_type=jnp.float32)
        out_ref[0] = shared * wsum + 0.1 * expert


@functools.partial(jax.jit, static_argnames=("interpret",))
def _run(x, Wup, Wgate, Wdown, Wpre, Wpost, g_norm, b_norm, Wadapt, adW,
         adg, adb, Wep, Wop, Wrg, Wre, interpret=False):
    wr = jnp.concatenate(
        [Wrg, Wre, jnp.zeros((D, 8 - NG - G), jnp.float32)], axis=1)
    adw_all = jnp.transpose(adW, (1, 0, 2)).reshape(A, E * A)
    gn = g_norm.reshape(1, A)
    bn = b_norm.reshape(1, A)
    ub = jnp.repeat(jnp.eye(E, dtype=jnp.float32), A, axis=0)
    bb = ub.T
    adgf = adg.reshape(1, E * A)
    wup_b = Wup.astype(jnp.bfloat16)
    wgate_b = Wgate.astype(jnp.bfloat16)
    wdown_b = Wdown.astype(jnp.bfloat16)
    wpost_b = Wpost.astype(jnp.bfloat16)

    wawd, wepwop = pl.pallas_call(
        _pre_body,
        out_shape=(jax.ShapeDtypeStruct((A, D), jnp.float32),
                   jax.ShapeDtypeStruct((A, D), jnp.float32)),
        interpret=interpret,
    )(Wadapt, Wdown, Wep, Wop)

    nt = S // TF
    out = pl.pallas_call(
        _fused_body,
        grid=(B, 2, nt),
        in_specs=[
            pl.BlockSpec((1, TF, D), lambda b, p, i: (b, i * (1 - p), 0)),
            pl.BlockSpec((D, H), lambda b, p, i: (0, 0)),
            pl.BlockSpec((D, H), lambda b, p, i: (0, 0)),
            pl.BlockSpec((H, D), lambda b, p, i: (0, 0)),
            pl.BlockSpec((D, A), lambda b, p, i: (0, 0)),
            pl.BlockSpec((H, A), lambda b, p, i: (0, 0)),
            pl.BlockSpec((1, A), lambda b, p, i: (0, 0)),
            pl.BlockSpec((1, A), lambda b, p, i: (0, 0)),
            pl.BlockSpec((D, 8), lambda b, p, i: (0, 0)),
            pl.BlockSpec((A, E * A), lambda b, p, i: (0, 0)),
            pl.BlockSpec((1, E * A), lambda b, p, i: (0, 0)),
            pl.BlockSpec((E, A), lambda b, p, i: (0, 0)),
            pl.BlockSpec((E * A, E), lambda b, p, i: (0, 0)),
            pl.BlockSpec((E, E * A), lambda b, p, i: (0, 0)),
            pl.BlockSpec((A, D), lambda b, p, i: (0, 0)),
            pl.BlockSpec((A, D), lambda b, p, i: (0, 0)),
        ],
        out_specs=pl.BlockSpec((1, TF, D), lambda b, p, i: (b, i * p, 0)),
        out_shape=jax.ShapeDtypeStruct((B, S, D), jnp.float32),
        scratch_shapes=[
            pltpu.VMEM((S, D), jnp.float32),
            pltpu.VMEM((S, A), jnp.float32),
            pltpu.VMEM((S, A), jnp.float32),
            pltpu.VMEM((S, A), jnp.float32),
            pltpu.VMEM((S, 8), jnp.float32),
        ],
        interpret=interpret,
    )(x, wup_b, wgate_b, wdown_b, Wpre, wpost_b, gn, bn, wr,
      adw_all, adgf, adb, ub, bb, wawd, wepwop)
    return out


def kernel(x, Wup, Wgate, Wdown, Wpre, Wpost, g_norm, b_norm, Wadapt, adW,
           adg, adb, Wep, Wop, Wrg, Wre):
    return _run(x, Wup, Wgate, Wdown, Wpre, Wpost, g_norm, b_norm, Wadapt,
                adW, adg, adb, Wep, Wop, Wrg, Wre)


# speedup vs baseline: 3.4606x; 1.0210x over previous
import functools

import jax
import jax.numpy as jnp
from jax.experimental import pallas as pl
from jax.experimental.pallas import tpu as pltpu

B, S, D = 2, 2048, 768
H = 2 * D
A = H // 16
E = 8
G = 4
NG = max(1, E // G)
K = 2
N = B * S
TF = 512


def _silu(x):
    return x * (0.5 * jnp.tanh(0.5 * x) + 0.5)


def _ln(x, g, b):
    mu = jnp.mean(x, axis=-1, keepdims=True)
    d = x - mu
    v = jnp.mean(d * d, axis=-1, keepdims=True)
    return d * jax.lax.rsqrt(v + 1e-5) * g + b


def _pre_body(wadapt_ref, wdown_ref, wep_ref, wop_ref, wawd_ref, wepwop_ref):
    wawd_ref[...] = jnp.dot(wadapt_ref[...], wdown_ref[...],
                            preferred_element_type=jnp.float32)
    wepwop_ref[...] = jnp.dot(wep_ref[...], wop_ref[...],
                              preferred_element_type=jnp.float32)


def _fused_body(x_ref, wup_ref, wgate_ref, wdown_ref, wpre_ref, wpost_ref,
                gn_ref, bn_ref, wr_ref,
                adw_ref, adgf_ref, adb_ref, ub_ref, bb_ref,
                wawd_ref, wepwop_ref,
                out_ref,
                sh_s, pre_s, ai_s, ao_s, rlog_s):
    p = pl.program_id(1)
    i = pl.program_id(2)
    rows = pl.ds(i * TF, TF)

    @pl.when(p == 0)
    def _main():
        xt = x_ref[0]
        xb = xt.astype(jnp.bfloat16)
        up = jnp.dot(xb, wup_ref[...], preferred_element_type=jnp.float32)
        gate = jnp.dot(xb, wgate_ref[...], preferred_element_type=jnp.float32)
        hidden = (_silu(gate) * up).astype(jnp.bfloat16)
        sh_s[rows, :] = jnp.dot(hidden, wdown_ref[...],
                                preferred_element_type=jnp.float32)
        pre = jnp.dot(xb, wpre_ref[...], preferred_element_type=jnp.float32)
        pre_s[rows, :] = pre
        g = gn_ref[...]
        b = bn_ref[...]
        ai_s[rows, :] = _ln(pre, g, b)
        po = jnp.dot(hidden, wpost_ref[...],
                     preferred_element_type=jnp.float32)
        ao_s[rows, :] = _ln(po, g, b)
        rlog_s[rows, :] = jnp.dot(xt, wr_ref[...],
                                  preferred_element_type=jnp.float32)

    @pl.when(p == 1)
    def _tail():
        ai_b = ai_s[...]
        ao_b = ao_s[...]
        ai_t = ai_s[rows, :]

        aw = jax.lax.dot_general(ai_t.astype(jnp.bfloat16),
                                 ao_b.astype(jnp.bfloat16),
                                 (((1,), (1,)), ((), ())),
                                 preferred_element_type=jnp.float32)
        aw = _silu(jnp.clip(aw, -5.0, 5.0)).astype(jnp.bfloat16)
        ac = jnp.dot(aw, ai_b.astype(jnp.bfloat16),
                     preferred_element_type=jnp.float32)
        adapt_contrib = jnp.dot(ac, wawd_ref[...],
                                preferred_element_type=jnp.float32)
        shared = sh_s[rows, :] + 0.1 * adapt_contrib

        rl = rlog_s[rows, :]
        gl0 = rl[:, 0:1]
        gl1 = rl[:, 1:2]
        gm = jnp.maximum(gl0, gl1)
        e0 = jnp.exp(gl0 - gm)
        e1 = jnp.exp(gl1 - gm)
        cw = jnp.maximum(e0, e1) / (e0 + e1)
        chosen = (gl1 > gl0).astype(jnp.int32)

        ll = rl[:, 2:2 + G]
        lm = jnp.max(ll, axis=1, keepdims=True)
        le = jnp.exp(ll - lm)
        lp = le / jnp.sum(le, axis=1, keepdims=True)
        idx4 = jax.lax.broadcasted_iota(jnp.int32, (TF, G), 1)
        p0 = jnp.max(lp, axis=1, keepdims=True)
        ti0 = jnp.min(jnp.where(lp == p0, idx4, G + 1), axis=1, keepdims=True)
        lp2 = jnp.where(idx4 == ti0, -jnp.inf, lp)
        p1 = jnp.max(lp2, axis=1, keepdims=True)
        ti1 = jnp.min(jnp.where(lp2 == p1, idx4, G + 1), axis=1, keepdims=True)
        inv = 1.0 / (p0 + p1 + 1e-7)
        fw0 = cw * p0 * inv
        fw1 = cw * p1 * inv
        wsum = fw0 + fw1

        g_of_e = jax.lax.broadcasted_iota(jnp.int32, (TF, E), 1) // G
        l_of_e = jax.lax.broadcasted_iota(jnp.int32, (TF, E), 1) % G
        dm = jnp.where(chosen == g_of_e, 1.0, 0.0) * (
            fw0 * jnp.where(ti0 == l_of_e, 1.0, 0.0)
            + fw1 * jnp.where(ti1 == l_of_e, 1.0, 0.0))

        pre_t = pre_s[rows, :]
        P = jnp.dot(pre_t, adw_ref[...],
                    preferred_element_type=jnp.float32)
        sums = jnp.dot(P, ub_ref[...], preferred_element_type=jnp.float32)
        sums2 = jnp.dot(P * P, ub_ref[...],
                        preferred_element_type=jnp.float32)
        mu = sums * (1.0 / A)
        var = sums2 * (1.0 / A) - mu * mu
        rs = jax.lax.rsqrt(var + 1e-5)
        sc = dm * rs
        uc = sc * mu
        Sb = jnp.dot(sc, bb_ref[...], preferred_element_type=jnp.float32)
        Ub = jnp.dot(uc, bb_ref[...], preferred_element_type=jnp.float32)
        zfull = (P * Sb - Ub) * adgf_ref[...]
        z = zfull[:, 0:A]
        for e in range(1, E):
            z = z + zfull[:, e * A:(e + 1) * A]
        z = z + jnp.dot(dm, adb_ref[...], preferred_element_type=jnp.float32)
        expert = jnp.dot(z, wepwop_ref[...],
                         preferred_element_type=jnp.float32)
        out_ref[0] = shared * wsum + 0.1 * expert


@functools.partial(jax.jit, static_argnames=("interpret",))
def _run(x, Wup, Wgate, Wdown, Wpre, Wpost, g_norm, b_norm, Wadapt, adW,
         adg, adb, Wep, Wop, Wrg, Wre, interpret=False):
    wr = jnp.concatenate(
        [Wrg, Wre, jnp.zeros((D, 8 - NG - G), jnp.float32)], axis=1)
    adw_all = jnp.transpose(adW, (1, 0, 2)).reshape(A, E * A)
    gn = g_norm.reshape(1, A)
    bn = b_norm.reshape(1, A)
    ub = jnp.repeat(jnp.eye(E, dtype=jnp.float32), A, axis=0)
    bb = ub.T
    adgf = adg.reshape(1, E * A)
    wup_b = Wup.astype(jnp.bfloat16)
    wgate_b = Wgate.astype(jnp.bfloat16)
    wdown_b = Wdown.astype(jnp.bfloat16)
    wpost_b = Wpost.astype(jnp.bfloat16)

    wawd, wepwop = pl.pallas_call(
        _pre_body,
        out_shape=(jax.ShapeDtypeStruct((A, D), jnp.float32),
                   jax.ShapeDtypeStruct((A, D), jnp.float32)),
        interpret=interpret,
    )(Wadapt, Wdown, Wep, Wop)

    nt = S // TF
    out = pl.pallas_call(
        _fused_body,
        grid=(B, 2, nt),
        in_specs=[
            pl.BlockSpec((1, TF, D), lambda b, p, i: (b, i * (1 - p), 0)),
            pl.BlockSpec((D, H), lambda b, p, i: (0, 0)),
            pl.BlockSpec((D, H), lambda b, p, i: (0, 0)),
            pl.BlockSpec((H, D), lambda b, p, i: (0, 0)),
            pl.BlockSpec((D, A), lambda b, p, i: (0, 0)),
            pl.BlockSpec((H, A), lambda b, p, i: (0, 0)),
            pl.BlockSpec((1, A), lambda b, p, i: (0, 0)),
            pl.BlockSpec((1, A), lambda b, p, i: (0, 0)),
            pl.BlockSpec((D, 8), lambda b, p, i: (0, 0)),
            pl.BlockSpec((A, E * A), lambda b, p, i: (0, 0)),
            pl.BlockSpec((1, E * A), lambda b, p, i: (0, 0)),
            pl.BlockSpec((E, A), lambda b, p, i: (0, 0)),
            pl.BlockSpec((E * A, E), lambda b, p, i: (0, 0)),
            pl.BlockSpec((E, E * A), lambda b, p, i: (0, 0)),
            pl.BlockSpec((A, D), lambda b, p, i: (0, 0)),
            pl.BlockSpec((A, D), lambda b, p, i: (0, 0)),
        ],
        out_specs=pl.BlockSpec((1, TF, D), lambda b, p, i: (b, i * p, 0)),
        out_shape=jax.ShapeDtypeStruct((B, S, D), jnp.float32),
        scratch_shapes=[
            pltpu.VMEM((S, D), jnp.float32),
            pltpu.VMEM((S, A), jnp.float32),
            pltpu.VMEM((S, A), jnp.float32),
            pltpu.VMEM((S, A), jnp.float32),
            pltpu.VMEM((S, 8), jnp.float32),
        ],
        interpret=interpret,
    )(x, wup_b, wgate_b, wdown_b, Wpre.astype(jnp.bfloat16), wpost_b, gn, bn,
      wr,
      adw_all, adgf, adb, ub, bb, wawd, wepwop)
    return out


def kernel(x, Wup, Wgate, Wdown, Wpre, Wpost, g_norm, b_norm, Wadapt, adW,
           adg, adb, Wep, Wop, Wrg, Wre):
    return _run(x, Wup, Wgate, Wdown, Wpre, Wpost, g_norm, b_norm, Wadapt,
                adW, adg, adb, Wep, Wop, Wrg, Wre)
